# Initial kernel scaffold; baseline (speedup 1.0000x reference)
import functools
import jax
import jax.numpy as jnp
from jax import lax
from jax.experimental import pallas as pl
from jax.experimental.pallas import tpu as pltpu
from jax.experimental.pallas import tpu_sc as plsc

DIM = 64
C = 128


def kernel(matrix, W_struct, W_value):
    B, L, K = matrix.shape
    NP = (B * L * K) // 2
    info = plsc.get_sparse_core_info()
    NC, NS = info.num_cores, info.num_subcores
    NW = NC * NS
    P = NP // NW
    n_chunks = P // C

    idx_flat = matrix.reshape(-1)

    mesh = plsc.VectorSubcoreMesh(core_axis_name="c", subcore_axis_name="s")

    @functools.partial(
        pl.kernel,
        mesh=mesh,
        out_type=jax.ShapeDtypeStruct((2 * NP, DIM), jnp.float32),
        scratch_types=[
            pltpu.VMEM((2 * C,), jnp.int32),
            pltpu.VMEM((C,), jnp.int32),
            pltpu.VMEM((C,), jnp.int32),
            pltpu.VMEM((C,), jnp.int32),
            pltpu.VMEM((C,), jnp.int32),
            pltpu.VMEM((C, DIM), jnp.float32),
            pltpu.VMEM((C, DIM), jnp.float32),
            pltpu.SemaphoreType.DMA,
            pltpu.SemaphoreType.DMA,
        ],
    )
    def k(idx_hbm, ws_hbm, wv_hbm, out_hbm,
          idx_iv, idx_s, idx_v, oe, oo, buf_s, buf_v, gsem, ssem):
        wid = lax.axis_index("s") * NC + lax.axis_index("c")
        pair_base = wid * P
        iota = lax.iota(jnp.int32, 16)

        def body(ci, carry):
            pb = pair_base + ci * C
            pltpu.sync_copy(idx_hbm.at[pl.ds(pb * 2, 2 * C)], idx_iv)

            def deint(j, carry2):
                even_pos = 2 * (iota + 16 * j)
                ev = plsc.load_gather(idx_iv, [even_pos])
                od = plsc.load_gather(idx_iv, [even_pos + 1])
                idx_s[pl.ds(16 * j, 16)] = ev
                idx_v[pl.ds(16 * j, 16)] = od
                rows = 2 * (pb + 16 * j + iota)
                oe[pl.ds(16 * j, 16)] = rows
                oo[pl.ds(16 * j, 16)] = rows + 1
                return carry2

            lax.fori_loop(0, C // 16, deint, 0)

            cp_s = pltpu.async_copy(ws_hbm.at[idx_s], buf_s, gsem)
            cp_v = pltpu.async_copy(wv_hbm.at[idx_v], buf_v, gsem)
            cp_s.wait()
            cp_v.wait()
            sc_s = pltpu.async_copy(buf_s, out_hbm.at[oe], ssem)
            sc_v = pltpu.async_copy(buf_v, out_hbm.at[oo], ssem)
            sc_s.wait()
            sc_v.wait()
            return carry

        lax.fori_loop(0, n_chunks, body, 0)

    out = k(idx_flat, W_struct, W_value)
    return out.reshape(B, L, K, DIM)


# speedup vs baseline: 4.4988x; 4.4988x over previous
import functools
import jax
import jax.numpy as jnp
from jax import lax
from jax.experimental import pallas as pl
from jax.experimental.pallas import tpu as pltpu
from jax.experimental.pallas import tpu_sc as plsc

DIM = 64
C = 128


def kernel(matrix, W_struct, W_value):
    B, L, K = matrix.shape
    NP = (B * L * K) // 2
    info = plsc.get_sparse_core_info()
    NC, NS = info.num_cores, info.num_subcores
    NW = NC * NS
    P = NP // NW
    n_chunks = P // C

    idx_flat = matrix.reshape(-1)

    mesh = plsc.VectorSubcoreMesh(core_axis_name="c", subcore_axis_name="s")

    @functools.partial(
        pl.kernel,
        mesh=mesh,
        compiler_params=pltpu.CompilerParams(
            needs_layout_passes=False, use_tc_tiling_on_sc=False),
        out_type=jax.ShapeDtypeStruct((2 * NP, DIM), jnp.float32),
        scratch_types=[
            pltpu.VMEM((2 * C,), jnp.int32),
            pltpu.VMEM((C,), jnp.int32),
            pltpu.VMEM((C,), jnp.int32),
            pltpu.VMEM((C,), jnp.int32),
            pltpu.VMEM((C,), jnp.int32),
            pltpu.VMEM((C, DIM), jnp.float32),
            pltpu.VMEM((C, DIM), jnp.float32),
            pltpu.SemaphoreType.DMA,
            pltpu.SemaphoreType.DMA,
        ],
    )
    def k(idx_hbm, ws_hbm, wv_hbm, out_hbm,
          idx_iv, idx_s, idx_v, oe, oo, buf_s, buf_v, gsem, ssem):
        wid = lax.axis_index("s") * NC + lax.axis_index("c")
        pair_base = wid * P
        iota = lax.iota(jnp.int32, 16)

        def body(ci, carry):
            pb = pair_base + ci * C
            pltpu.sync_copy(idx_hbm.at[pl.ds(pb * 2, 2 * C)], idx_iv)

            def deint(j, carry2):
                even_pos = 2 * (iota + 16 * j)
                ev = plsc.load_gather(idx_iv, [even_pos])
                od = plsc.load_gather(idx_iv, [even_pos + 1])
                idx_s[pl.ds(16 * j, 16)] = ev
                idx_v[pl.ds(16 * j, 16)] = od
                rows = 2 * (pb + 16 * j + iota)
                oe[pl.ds(16 * j, 16)] = rows
                oo[pl.ds(16 * j, 16)] = rows + 1
                return carry2

            lax.fori_loop(0, C // 16, deint, 0)

            cp_s = pltpu.async_copy(ws_hbm.at[idx_s], buf_s, gsem)
            cp_v = pltpu.async_copy(wv_hbm.at[idx_v], buf_v, gsem)
            cp_s.wait()
            cp_v.wait()
            sc_s = pltpu.async_copy(buf_s, out_hbm.at[oe], ssem)
            sc_v = pltpu.async_copy(buf_v, out_hbm.at[oo], ssem)
            sc_s.wait()
            sc_v.wait()
            return carry

        lax.fori_loop(0, n_chunks, body, 0)

    out = k(idx_flat, W_struct, W_value)
    return out.reshape(B, L, K, DIM)


# speedup vs baseline: 4.7365x; 1.0528x over previous
import functools
import jax
import jax.numpy as jnp
from jax import lax
from jax.experimental import pallas as pl
from jax.experimental.pallas import tpu as pltpu
from jax.experimental.pallas import tpu_sc as plsc

DIM = 64
C = 64


def kernel(matrix, W_struct, W_value):
    B, L, K = matrix.shape
    NP = (B * L * K) // 2
    info = plsc.get_sparse_core_info()
    NC, NS = info.num_cores, info.num_subcores
    NW = NC * NS
    P = NP // NW
    n_chunks = P // C

    idx_flat = matrix.reshape(-1)

    mesh = plsc.VectorSubcoreMesh(core_axis_name="c", subcore_axis_name="s")

    @functools.partial(
        pl.kernel,
        mesh=mesh,
        compiler_params=pltpu.CompilerParams(
            needs_layout_passes=False, use_tc_tiling_on_sc=False),
        out_type=jax.ShapeDtypeStruct((2 * NP, DIM), jnp.float32),
        scratch_types=[
            pltpu.VMEM((2 * C,), jnp.int32),
            pltpu.VMEM((2, C), jnp.int32),
            pltpu.VMEM((2, C), jnp.int32),
            pltpu.VMEM((2, C), jnp.int32),
            pltpu.VMEM((2, C), jnp.int32),
            pltpu.VMEM((2, C, DIM), jnp.float32),
            pltpu.VMEM((2, C, DIM), jnp.float32),
            pltpu.SemaphoreType.DMA,
            pltpu.SemaphoreType.DMA,
            pltpu.SemaphoreType.DMA,
            pltpu.SemaphoreType.DMA,
        ],
    )
    def k(idx_hbm, ws_hbm, wv_hbm, out_hbm,
          idx_iv, idx_s, idx_v, oe, oo, buf_s, buf_v,
          gsem0, gsem1, ssem0, ssem1):
        wid = lax.axis_index("s") * NC + lax.axis_index("c")
        pair_base = wid * P
        iota = lax.iota(jnp.int32, 16)
        gsem = (gsem0, gsem1)
        ssem = (ssem0, ssem1)

        def load_deint(ci, b):
            pb = pair_base + ci * C
            pltpu.sync_copy(idx_hbm.at[pl.ds(pb * 2, 2 * C)], idx_iv)
            for j in range(C // 16):
                even_pos = 2 * (iota + 16 * j)
                ev = plsc.load_gather(idx_iv, [even_pos])
                od = plsc.load_gather(idx_iv, [even_pos + 1])
                idx_s[b, pl.ds(16 * j, 16)] = ev
                idx_v[b, pl.ds(16 * j, 16)] = od
                rows = 2 * (pb + 16 * j + iota)
                oe[b, pl.ds(16 * j, 16)] = rows
                oo[b, pl.ds(16 * j, 16)] = rows + 1

        def start_gather(b):
            pltpu.async_copy(ws_hbm.at[idx_s.at[b]], buf_s.at[b], gsem[b])
            pltpu.async_copy(wv_hbm.at[idx_v.at[b]], buf_v.at[b], gsem[b])

        def wait_gather(b):
            pltpu.make_async_copy(ws_hbm.at[idx_s.at[b]], buf_s.at[b], gsem[b]).wait()
            pltpu.make_async_copy(wv_hbm.at[idx_v.at[b]], buf_v.at[b], gsem[b]).wait()

        def start_scatter(b):
            pltpu.async_copy(buf_s.at[b], out_hbm.at[oe.at[b]], ssem[b])
            pltpu.async_copy(buf_v.at[b], out_hbm.at[oo.at[b]], ssem[b])

        def wait_scatter(b):
            pltpu.make_async_copy(buf_s.at[b], out_hbm.at[oe.at[b]], ssem[b]).wait()
            pltpu.make_async_copy(buf_v.at[b], out_hbm.at[oo.at[b]], ssem[b]).wait()

        load_deint(0, 0)
        start_gather(0)
        load_deint(1, 1)
        start_gather(1)
        wait_gather(0)
        start_scatter(0)

        @pl.loop(2, n_chunks, step=2)
        def _(ci0):
            for bb in range(2):
                ci = ci0 + bb
                b = bb
                wait_scatter(b)
                load_deint(ci, b)
                start_gather(b)
                wait_gather(1 - b)
                start_scatter(1 - b)

        last = (n_chunks - 1) % 2
        wait_gather(last)
        start_scatter(last)
        wait_scatter(1 - last)
        wait_scatter(last)

    out = k(idx_flat, W_struct, W_value)
    return out.reshape(B, L, K, DIM)
